# baseline (speedup 1.0000x reference)
import jax
import jax.numpy as jnp
from jax import lax
from jax.experimental import pallas as pl
from jax.experimental.pallas import tpu as pltpu
from jax.experimental.pallas import tpu_sc as plsc

BATCH = 8192
D = 2048
NC = 2
NS = 16
NW = NC * NS
L = 16
ROWS_PER_W = BATCH // NW
CHUNK = 8
NCHUNKS = ROWS_PER_W // CHUNK
GROUPS = D // L


def _permute_body(in_hbm, perm_hbm, out_hbm, perm_v, in_v, out_v, sem):
    wid = lax.axis_index("s") * NC + lax.axis_index("c")
    base = wid * ROWS_PER_W * D

    pltpu.sync_copy(perm_hbm, perm_v)

    def chunk_body(c, _):
        off = base + c * (CHUNK * D)
        pltpu.async_copy(in_hbm.at[pl.ds(off, CHUNK * D)], in_v, sem).wait()
        for r in range(CHUNK):
            def group_body(g, _):
                idx = perm_v[pl.ds(g * L, L)] + (r * D)
                vals = plsc.load_gather(in_v, [idx])
                out_v[pl.ds(r * D + g * L, L)] = vals
                return 0

            lax.fori_loop(0, GROUPS, group_body, 0, unroll=8)
        pltpu.async_copy(out_v, out_hbm.at[pl.ds(off, CHUNK * D)], sem).wait()
        return 0

    lax.fori_loop(0, NCHUNKS, chunk_body, 0)


@jax.jit
def _permute(inputs_flat, perm):
    mesh = plsc.VectorSubcoreMesh(core_axis_name="c", subcore_axis_name="s")
    return pl.kernel(
        _permute_body,
        mesh=mesh,
        out_type=jax.ShapeDtypeStruct((BATCH * D,), jnp.float32),
        scratch_types=[
            pltpu.VMEM((D,), jnp.int32),
            pltpu.VMEM((CHUNK * D,), jnp.float32),
            pltpu.VMEM((CHUNK * D,), jnp.float32),
            pltpu.SemaphoreType.DMA,
        ],
        compiler_params=pltpu.CompilerParams(needs_layout_passes=False),
    )(inputs_flat, perm)


def kernel(inputs, perm):
    out = _permute(inputs.reshape(-1), perm.astype(jnp.int32))
    logdet = jnp.zeros(inputs.shape[:1], dtype=inputs.dtype)
    return (out.reshape(BATCH, D), logdet)


# speedup vs baseline: 2.8009x; 2.8009x over previous
import jax
import jax.numpy as jnp
from jax import lax
from jax.experimental import pallas as pl
from jax.experimental.pallas import tpu as pltpu
from jax.experimental.pallas import tpu_sc as plsc

BATCH = 8192
D = 2048
NC = 2
NS = 16
NW = NC * NS
L = 16
ROWS_PER_W = BATCH // NW
CHUNK = 8
CPD = CHUNK * D
NCHUNKS = ROWS_PER_W // CHUNK
NPAIRS = NCHUNKS // 2
GROUPS = D // L
UNROLL = 4


def _permute_body(in_hbm, perm_hbm, out_hbm,
                  perm_v, in_v0, in_v1, out_v0, out_v1,
                  sem_i0, sem_i1, sem_o0, sem_o1):
    wid = lax.axis_index("s") * NC + lax.axis_index("c")
    base = wid * ROWS_PER_W * D

    pltpu.sync_copy(perm_hbm, perm_v)

    def start_in(c, buf, sem):
        pltpu.async_copy(in_hbm.at[pl.ds(base + c * CPD, CPD)], buf, sem)

    def start_out(c, buf, sem):
        pltpu.async_copy(buf, out_hbm.at[pl.ds(base + c * CPD, CPD)], sem)

    def wait(buf, sem):
        pltpu.make_async_copy(in_hbm.at[pl.ds(0, CPD)], buf, sem).wait()

    def gather_chunk(src, dst):
        @plsc.parallel_loop(0, GROUPS, unroll=UNROLL)
        def _(g):
            gl = g * L
            idx = perm_v[pl.ds(gl, L)]
            for r in range(CHUNK):
                vals = plsc.load_gather(src.at[pl.ds(r * D, D)], [idx])
                dst[pl.ds(r * D + gl, L)] = vals

    start_in(0, in_v0, sem_i0)

    def pair_body(p, _):
        c0 = p * 2
        c1 = c0 + 1
        start_in(c1, in_v1, sem_i1)
        wait(in_v0, sem_i0)

        @pl.when(p > 0)
        def _():
            wait(out_v0, sem_o0)
        gather_chunk(in_v0, out_v0)
        start_out(c0, out_v0, sem_o0)

        @pl.when(p < NPAIRS - 1)
        def _():
            start_in(c0 + 2, in_v0, sem_i0)
        wait(in_v1, sem_i1)

        @pl.when(p > 0)
        def _():
            wait(out_v1, sem_o1)
        gather_chunk(in_v1, out_v1)
        start_out(c1, out_v1, sem_o1)
        return 0

    lax.fori_loop(0, NPAIRS, pair_body, 0)

    wait(out_v0, sem_o0)
    wait(out_v1, sem_o1)


@jax.jit
def _permute(inputs_flat, perm):
    mesh = plsc.VectorSubcoreMesh(core_axis_name="c", subcore_axis_name="s")
    return pl.kernel(
        _permute_body,
        mesh=mesh,
        out_type=jax.ShapeDtypeStruct((BATCH * D,), jnp.float32),
        scratch_types=[
            pltpu.VMEM((D,), jnp.int32),
            pltpu.VMEM((CPD,), jnp.float32),
            pltpu.VMEM((CPD,), jnp.float32),
            pltpu.VMEM((CPD,), jnp.float32),
            pltpu.VMEM((CPD,), jnp.float32),
            pltpu.SemaphoreType.DMA,
            pltpu.SemaphoreType.DMA,
            pltpu.SemaphoreType.DMA,
            pltpu.SemaphoreType.DMA,
        ],
        compiler_params=pltpu.CompilerParams(needs_layout_passes=False),
    )(inputs_flat, perm)


def kernel(inputs, perm):
    out = _permute(inputs.reshape(-1), perm.astype(jnp.int32))
    logdet = jnp.zeros(inputs.shape[:1], dtype=inputs.dtype)
    return (out.reshape(BATCH, D), logdet)


# speedup vs baseline: 7.3603x; 2.6279x over previous
import jax
import jax.numpy as jnp
from jax import lax
from jax.experimental import pallas as pl
from jax.experimental.pallas import tpu as pltpu
from jax.experimental.pallas import tpu_sc as plsc

BATCH = 8192
D = 2048
NC = 2
NS = 16
NW = NC * NS
L = 16
ROWS_PER_W = BATCH // NW
CHUNK = 8
CPD = CHUNK * D
NCHUNKS = ROWS_PER_W // CHUNK
NPAIRS = NCHUNKS // 2
GROUPS = D // L
UNROLL = 4


def _permute_body(in_hbm, perm_hbm, out_hbm,
                  perm_v, in_v0, in_v1, out_v0, out_v1,
                  sem_i0, sem_i1, sem_o0, sem_o1):
    wid = lax.axis_index("s") * NC + lax.axis_index("c")
    base_row = wid * ROWS_PER_W

    pltpu.sync_copy(perm_hbm, perm_v)

    def start_in(c, buf, sem):
        row0 = base_row + c * CHUNK
        for k in range(CHUNK):
            pltpu.async_copy(
                in_hbm.at[row0 + k, :], buf.at[pl.ds(k * D, D)], sem)

    def start_out(c, buf, sem):
        row0 = base_row + c * CHUNK
        for k in range(CHUNK):
            pltpu.async_copy(
                buf.at[pl.ds(k * D, D)], out_hbm.at[row0 + k, :], sem)

    def wait(buf, sem):
        for k in range(CHUNK):
            pltpu.make_async_copy(
                in_hbm.at[0, :], buf.at[pl.ds(0, D)], sem).wait()

    def gather_chunk(src, dst):
        @plsc.parallel_loop(0, GROUPS, unroll=UNROLL)
        def _(g):
            gl = g * L
            idx = perm_v[pl.ds(gl, L)]
            for r in range(CHUNK):
                vals = plsc.load_gather(src.at[pl.ds(r * D, D)], [idx])
                dst[pl.ds(r * D + gl, L)] = vals

    start_in(0, in_v0, sem_i0)

    def pair_body(p, _):
        c0 = p * 2
        c1 = c0 + 1
        start_in(c1, in_v1, sem_i1)
        wait(in_v0, sem_i0)

        @pl.when(p > 0)
        def _():
            wait(out_v0, sem_o0)
        gather_chunk(in_v0, out_v0)
        start_out(c0, out_v0, sem_o0)

        @pl.when(p < NPAIRS - 1)
        def _():
            start_in(c0 + 2, in_v0, sem_i0)
        wait(in_v1, sem_i1)

        @pl.when(p > 0)
        def _():
            wait(out_v1, sem_o1)
        gather_chunk(in_v1, out_v1)
        start_out(c1, out_v1, sem_o1)
        return 0

    lax.fori_loop(0, NPAIRS, pair_body, 0)

    wait(out_v0, sem_o0)
    wait(out_v1, sem_o1)


@jax.jit
def _permute(inputs, perm):
    mesh = plsc.VectorSubcoreMesh(core_axis_name="c", subcore_axis_name="s")
    return pl.kernel(
        _permute_body,
        mesh=mesh,
        out_type=jax.ShapeDtypeStruct((BATCH, D), jnp.float32),
        scratch_types=[
            pltpu.VMEM((D,), jnp.int32),
            pltpu.VMEM((CPD,), jnp.float32),
            pltpu.VMEM((CPD,), jnp.float32),
            pltpu.VMEM((CPD,), jnp.float32),
            pltpu.VMEM((CPD,), jnp.float32),
            pltpu.SemaphoreType.DMA,
            pltpu.SemaphoreType.DMA,
            pltpu.SemaphoreType.DMA,
            pltpu.SemaphoreType.DMA,
        ],
        compiler_params=pltpu.CompilerParams(needs_layout_passes=False),
    )(inputs, perm)


def kernel(inputs, perm):
    out = _permute(inputs, perm.astype(jnp.int32))
    logdet = jnp.zeros(inputs.shape[:1], dtype=inputs.dtype)
    return (out, logdet)


# speedup vs baseline: 7.4159x; 1.0076x over previous
import jax
import jax.numpy as jnp
from jax import lax
from jax.experimental import pallas as pl
from jax.experimental.pallas import tpu as pltpu
from jax.experimental.pallas import tpu_sc as plsc

BATCH = 8192
D = 2048
NC = 2
NS = 16
NW = NC * NS
L = 16
ROWS_PER_W = BATCH // NW
CHUNK = 8
CPD = CHUNK * D
NCHUNKS = ROWS_PER_W // CHUNK
NPAIRS = NCHUNKS // 2
GROUPS = D // L
UNROLL = 4


def _permute_body(in_hbm, perm_hbm, out_hbm,
                  perm_v, in_v0, in_v1, out_v0, out_v1,
                  sem_i0, sem_i1, sem_o0, sem_o1):
    wid = lax.axis_index("s") * NC + lax.axis_index("c")
    base_row = wid * ROWS_PER_W

    pltpu.sync_copy(perm_hbm, perm_v)

    def start_in(c, buf, sem):
        pltpu.async_copy(
            in_hbm.at[pl.ds(base_row + c * CHUNK, CHUNK), :], buf, sem)

    def start_out(c, buf, sem):
        pltpu.async_copy(
            buf, out_hbm.at[pl.ds(base_row + c * CHUNK, CHUNK), :], sem)

    def wait(buf, sem):
        pltpu.make_async_copy(in_hbm.at[pl.ds(0, CHUNK), :], buf, sem).wait()

    def gather_chunk(src, dst):
        rvecs = [jnp.full((L,), r, dtype=jnp.int32) for r in range(CHUNK)]

        @plsc.parallel_loop(0, GROUPS, unroll=UNROLL)
        def _(g):
            gl = g * L
            idx = perm_v[pl.ds(gl, L)]
            for r in range(CHUNK):
                vals = plsc.load_gather(src, [rvecs[r], idx])
                dst[r, pl.ds(gl, L)] = vals

    start_in(0, in_v0, sem_i0)

    def pair_body(p, _):
        c0 = p * 2
        c1 = c0 + 1
        start_in(c1, in_v1, sem_i1)
        wait(in_v0, sem_i0)

        @pl.when(p > 0)
        def _():
            wait(out_v0, sem_o0)
        gather_chunk(in_v0, out_v0)
        start_out(c0, out_v0, sem_o0)

        @pl.when(p < NPAIRS - 1)
        def _():
            start_in(c0 + 2, in_v0, sem_i0)
        wait(in_v1, sem_i1)

        @pl.when(p > 0)
        def _():
            wait(out_v1, sem_o1)
        gather_chunk(in_v1, out_v1)
        start_out(c1, out_v1, sem_o1)
        return 0

    lax.fori_loop(0, NPAIRS, pair_body, 0)

    wait(out_v0, sem_o0)
    wait(out_v1, sem_o1)


@jax.jit
def _permute(inputs, perm):
    mesh = plsc.VectorSubcoreMesh(core_axis_name="c", subcore_axis_name="s")
    return pl.kernel(
        _permute_body,
        mesh=mesh,
        out_type=jax.ShapeDtypeStruct((BATCH, D), jnp.float32),
        scratch_types=[
            pltpu.VMEM((D,), jnp.int32),
            pltpu.VMEM((CHUNK, D), jnp.float32),
            pltpu.VMEM((CHUNK, D), jnp.float32),
            pltpu.VMEM((CHUNK, D), jnp.float32),
            pltpu.VMEM((CHUNK, D), jnp.float32),
            pltpu.SemaphoreType.DMA,
            pltpu.SemaphoreType.DMA,
            pltpu.SemaphoreType.DMA,
            pltpu.SemaphoreType.DMA,
        ],
        compiler_params=pltpu.CompilerParams(needs_layout_passes=False),
    )(inputs, perm)


def kernel(inputs, perm):
    out = _permute(inputs, perm.astype(jnp.int32))
    logdet = jnp.zeros(inputs.shape[:1], dtype=inputs.dtype)
    return (out, logdet)


# speedup vs baseline: 7.5277x; 1.0151x over previous
import jax
import jax.numpy as jnp
from jax import lax
from jax.experimental import pallas as pl
from jax.experimental.pallas import tpu as pltpu
from jax.experimental.pallas import tpu_sc as plsc

BATCH = 8192
D = 2048
NC = 2
NS = 16
NW = NC * NS
L = 16
ROWS_PER_W = BATCH // NW
CHUNK = 8
CPD = CHUNK * D
NCHUNKS = ROWS_PER_W // CHUNK
NPAIRS = NCHUNKS // 2
GROUPS = D // L
UNROLL = 8


def _permute_body(in_hbm, perm_hbm, out_hbm,
                  perm_v, in_v0, in_v1, out_v0, out_v1,
                  sem_i0, sem_i1, sem_o0, sem_o1):
    wid = lax.axis_index("s") * NC + lax.axis_index("c")
    base_row = wid * ROWS_PER_W

    def start_in(c, buf, sem):
        pltpu.async_copy(
            in_hbm.at[pl.ds(base_row + c * CHUNK, CHUNK), :], buf, sem)

    def start_out(c, buf, sem):
        pltpu.async_copy(
            buf, out_hbm.at[pl.ds(base_row + c * CHUNK, CHUNK), :], sem)

    def wait(buf, sem):
        pltpu.make_async_copy(in_hbm.at[pl.ds(0, CHUNK), :], buf, sem).wait()

    def gather_chunk(src, dst):
        rvecs = [jnp.full((L,), r, dtype=jnp.int32) for r in range(CHUNK)]

        @plsc.parallel_loop(0, GROUPS, unroll=UNROLL)
        def _(g):
            gl = g * L
            idx = perm_v[pl.ds(gl, L)]
            for r in range(CHUNK):
                vals = plsc.load_gather(src, [rvecs[r], idx])
                dst[r, pl.ds(gl, L)] = vals

    start_in(0, in_v0, sem_i0)
    pltpu.sync_copy(perm_hbm, perm_v)

    def pair_body(p, _):
        c0 = p * 2
        c1 = c0 + 1
        start_in(c1, in_v1, sem_i1)
        wait(in_v0, sem_i0)

        @pl.when(p > 0)
        def _():
            wait(out_v0, sem_o0)
        gather_chunk(in_v0, out_v0)
        start_out(c0, out_v0, sem_o0)

        @pl.when(p < NPAIRS - 1)
        def _():
            start_in(c0 + 2, in_v0, sem_i0)
        wait(in_v1, sem_i1)

        @pl.when(p > 0)
        def _():
            wait(out_v1, sem_o1)
        gather_chunk(in_v1, out_v1)
        start_out(c1, out_v1, sem_o1)
        return 0

    lax.fori_loop(0, NPAIRS, pair_body, 0)

    wait(out_v0, sem_o0)
    wait(out_v1, sem_o1)


@jax.jit
def _permute(inputs, perm):
    mesh = plsc.VectorSubcoreMesh(core_axis_name="c", subcore_axis_name="s")
    return pl.kernel(
        _permute_body,
        mesh=mesh,
        out_type=jax.ShapeDtypeStruct((BATCH, D), jnp.float32),
        scratch_types=[
            pltpu.VMEM((D,), jnp.int32),
            pltpu.VMEM((CHUNK, D), jnp.float32),
            pltpu.VMEM((CHUNK, D), jnp.float32),
            pltpu.VMEM((CHUNK, D), jnp.float32),
            pltpu.VMEM((CHUNK, D), jnp.float32),
            pltpu.SemaphoreType.DMA,
            pltpu.SemaphoreType.DMA,
            pltpu.SemaphoreType.DMA,
            pltpu.SemaphoreType.DMA,
        ],
        compiler_params=pltpu.CompilerParams(needs_layout_passes=False),
    )(inputs, perm)


def kernel(inputs, perm):
    out = _permute(inputs, perm.astype(jnp.int32))
    logdet = jnp.zeros(inputs.shape[:1], dtype=inputs.dtype)
    return (out, logdet)


# speedup vs baseline: 7.7098x; 1.0242x over previous
import jax
import jax.numpy as jnp
from jax import lax
from jax.experimental import pallas as pl
from jax.experimental.pallas import tpu as pltpu
from jax.experimental.pallas import tpu_sc as plsc

BATCH = 8192
D = 2048
NC = 2
NS = 16
NW = NC * NS
L = 16
ROWS_PER_W = BATCH // NW
CHUNK = 4
NBUF = 4
NCHUNKS = ROWS_PER_W // CHUNK
NQUADS = NCHUNKS // NBUF
GROUPS = D // L
UNROLL = 8


def _permute_body(in_hbm, perm_hbm, out_hbm,
                  perm_v,
                  in_b0, in_b1, in_b2, in_b3,
                  out_b0, out_b1, out_b2, out_b3,
                  sem_i0, sem_i1, sem_i2, sem_i3,
                  sem_o0, sem_o1, sem_o2, sem_o3):
    in_bufs = (in_b0, in_b1, in_b2, in_b3)
    out_bufs = (out_b0, out_b1, out_b2, out_b3)
    sem_is = (sem_i0, sem_i1, sem_i2, sem_i3)
    sem_os = (sem_o0, sem_o1, sem_o2, sem_o3)

    wid = lax.axis_index("s") * NC + lax.axis_index("c")
    base_row = wid * ROWS_PER_W

    def start_in(c, buf, sem):
        pltpu.async_copy(
            in_hbm.at[pl.ds(base_row + c * CHUNK, CHUNK), :], buf, sem)

    def start_out(c, buf, sem):
        pltpu.async_copy(
            buf, out_hbm.at[pl.ds(base_row + c * CHUNK, CHUNK), :], sem)

    def wait(buf, sem):
        pltpu.make_async_copy(in_hbm.at[pl.ds(0, CHUNK), :], buf, sem).wait()

    def gather_chunk(src, dst):
        rvecs = [jnp.full((L,), r, dtype=jnp.int32) for r in range(CHUNK)]

        @plsc.parallel_loop(0, GROUPS, unroll=UNROLL)
        def _(g):
            gl = g * L
            idx = perm_v[pl.ds(gl, L)]
            for r in range(CHUNK):
                vals = plsc.load_gather(src, [rvecs[r], idx])
                dst[r, pl.ds(gl, L)] = vals

    for b in range(NBUF):
        start_in(b, in_bufs[b], sem_is[b])
    pltpu.sync_copy(perm_hbm, perm_v)

    def quad_body(q, _):
        c0 = q * NBUF
        for b in range(NBUF):
            c = c0 + b
            wait(in_bufs[b], sem_is[b])

            @pl.when(q > 0)
            def _():
                wait(out_bufs[b], sem_os[b])
            gather_chunk(in_bufs[b], out_bufs[b])
            start_out(c, out_bufs[b], sem_os[b])

            @pl.when(q < NQUADS - 1)
            def _():
                start_in(c + NBUF, in_bufs[b], sem_is[b])
        return 0

    lax.fori_loop(0, NQUADS, quad_body, 0)

    for b in range(NBUF):
        wait(out_bufs[b], sem_os[b])


@jax.jit
def _permute(inputs, perm):
    mesh = plsc.VectorSubcoreMesh(core_axis_name="c", subcore_axis_name="s")
    return pl.kernel(
        _permute_body,
        mesh=mesh,
        out_type=jax.ShapeDtypeStruct((BATCH, D), jnp.float32),
        scratch_types=[
            pltpu.VMEM((D,), jnp.int32),
            pltpu.VMEM((CHUNK, D), jnp.float32),
            pltpu.VMEM((CHUNK, D), jnp.float32),
            pltpu.VMEM((CHUNK, D), jnp.float32),
            pltpu.VMEM((CHUNK, D), jnp.float32),
            pltpu.VMEM((CHUNK, D), jnp.float32),
            pltpu.VMEM((CHUNK, D), jnp.float32),
            pltpu.VMEM((CHUNK, D), jnp.float32),
            pltpu.VMEM((CHUNK, D), jnp.float32),
            pltpu.SemaphoreType.DMA,
            pltpu.SemaphoreType.DMA,
            pltpu.SemaphoreType.DMA,
            pltpu.SemaphoreType.DMA,
            pltpu.SemaphoreType.DMA,
            pltpu.SemaphoreType.DMA,
            pltpu.SemaphoreType.DMA,
            pltpu.SemaphoreType.DMA,
        ],
        compiler_params=pltpu.CompilerParams(needs_layout_passes=False),
    )(inputs, perm)


def kernel(inputs, perm):
    out = _permute(inputs, perm.astype(jnp.int32))
    logdet = jnp.zeros(inputs.shape[:1], dtype=inputs.dtype)
    return (out, logdet)
